# baseline (speedup 1.0000x reference)
import functools

import jax
import jax.numpy as jnp
from jax import lax
from jax.experimental import pallas as pl
from jax.experimental.pallas import tpu as pltpu
from jax.experimental.pallas import tpu_sc as plsc

_NOBS = 1000000
_ROWS = 16384
_COLS = 200
_NC = 2
_NS = 16
_NW = _NC * _NS
_RPW = _ROWS // _NW
_RCH = 16
_CHUNK = _RCH * _COLS
_NCHUNK = _RPW // _RCH
_STAGE = 62496
_SCH = 10416
_L = 16
_TC0 = _COLS - _L

_mesh = plsc.VectorSubcoreMesh(core_axis_name="c", subcore_axis_name="s")


@functools.partial(
    pl.kernel,
    out_type=jax.ShapeDtypeStruct((_ROWS, _COLS), jnp.float32),
    mesh=_mesh,
    compiler_params=pltpu.CompilerParams(needs_layout_passes=False),
    scratch_types=[
        pltpu.VMEM_SHARED((_NOBS,), jnp.float32),
        [pltpu.VMEM((_RCH, _COLS), jnp.int32) for _ in range(2)],
        [pltpu.VMEM((_CHUNK,), jnp.int32) for _ in range(2)],
        [pltpu.VMEM((_CHUNK,), jnp.float32) for _ in range(2)],
        [pltpu.VMEM((_RCH, _COLS), jnp.float32) for _ in range(2)],
        pltpu.VMEM((_SCH,), jnp.float32),
        pltpu.SemaphoreType.DMA((2,)),
        pltpu.SemaphoreType.DMA((2,)),
        pltpu.SemaphoreType.DMA((2,)),
    ],
)
def _gather_kernel(idx_hbm, table_hbm, out_hbm, tbl_s, idx2_v, idx1_v,
                   out1_v, out2_v, stg_v, lsem, gsem, ssem):
    s = lax.axis_index("s")
    wid = s * _NC + lax.axis_index("c")
    base = wid * _RPW
    lane = lax.iota(jnp.int32, _L)

    def ichunk(i):
        return idx_hbm.at[pl.ds(base + i * _RCH, _RCH), :]

    def ochunk(i):
        return out_hbm.at[pl.ds(base + i * _RCH, _RCH), :]

    def compact(src2, dst1):
        def body(r, carry):
            for c in range(0, _TC0, _L):
                dst1[pl.ds(r * _COLS + c, _L)] = src2[r, pl.ds(c, _L)]
            rv = jnp.full((_L,), r, dtype=jnp.int32)
            tail = plsc.load_gather(src2, [rv, lane + _TC0])
            dst1[pl.ds(r * _COLS + _TC0, _L)] = tail
            return carry
        lax.fori_loop(0, _RCH, body, 0)

    def expand(src1, dst2):
        def body(r, carry):
            for c in range(0, _TC0, _L):
                dst2[r, pl.ds(c, _L)] = src1[pl.ds(r * _COLS + c, _L)]
            rv = jnp.full((_L,), r, dtype=jnp.int32)
            tail = src1[pl.ds(r * _COLS + _TC0, _L)]
            plsc.store_scatter(dst2, [rv, lane + _TC0], tail)
            return carry
        lax.fori_loop(0, _RCH, body, 0)

    def drain_gather(b):
        pltpu.make_async_copy(
            table_hbm.at[pl.ds(0, _CHUNK)], out1_v[b], gsem.at[b]).wait()

    pltpu.async_copy(ichunk(0), idx2_v[0], lsem.at[0])
    pltpu.async_copy(ichunk(1), idx2_v[1], lsem.at[1])
    pltpu.make_async_copy(ichunk(0), idx2_v[0], lsem.at[0]).wait()
    compact(idx2_v[0], idx1_v[0])
    pltpu.async_copy(table_hbm.at[idx1_v[0]], out1_v[0], gsem.at[0])
    pltpu.async_copy(ichunk(2), idx2_v[0], lsem.at[0])
    pltpu.make_async_copy(ichunk(1), idx2_v[1], lsem.at[1]).wait()
    compact(idx2_v[1], idx1_v[1])
    pltpu.async_copy(table_hbm.at[idx1_v[1]], out1_v[1], gsem.at[1])
    pltpu.async_copy(ichunk(3), idx2_v[1], lsem.at[1])

    for j in range(_STAGE // _SCH):
        soff = s * _STAGE + j * _SCH
        pltpu.sync_copy(table_hbm.at[pl.ds(soff, _SCH)], stg_v)
        pltpu.sync_copy(stg_v, tbl_s.at[pl.ds(soff, _SCH)])

    @pl.when(s == 0)
    def _():
        rem = _NOBS - _NS * _STAGE
        roff = _NS * _STAGE
        bv = stg_v.at[pl.ds(0, rem)]
        pltpu.sync_copy(table_hbm.at[pl.ds(roff, rem)], bv)
        pltpu.sync_copy(bv, tbl_s.at[pl.ds(roff, rem)])

    plsc.subcore_barrier()

    for i in range(2, _NCHUNK):
        b = i % 2
        drain_gather(b)
        if i >= 4:
            pltpu.make_async_copy(
                out2_v[b], ochunk(i - 4), ssem.at[b]).wait()
        expand(out1_v[b], out2_v[b])
        pltpu.async_copy(out2_v[b], ochunk(i - 2), ssem.at[b])
        pltpu.make_async_copy(ichunk(i), idx2_v[b], lsem.at[b]).wait()
        compact(idx2_v[b], idx1_v[b])
        pltpu.async_copy(tbl_s.at[idx1_v[b]], out1_v[b], gsem.at[b])
        if i + 2 < _NCHUNK:
            pltpu.async_copy(ichunk(i + 2), idx2_v[b], lsem.at[b])

    for k in (_NCHUNK - 2, _NCHUNK - 1):
        b = k % 2
        drain_gather(b)
        pltpu.make_async_copy(out2_v[b], ochunk(k - 2), ssem.at[b]).wait()
        expand(out1_v[b], out2_v[b])
        pltpu.async_copy(out2_v[b], ochunk(k), ssem.at[b])
    for k in (_NCHUNK - 2, _NCHUNK - 1):
        b = k % 2
        pltpu.make_async_copy(out2_v[b], ochunk(k), ssem.at[b]).wait()


def kernel(iobs, cloud_fraction_delta):
    return _gather_kernel(iobs, cloud_fraction_delta)


# speedup vs baseline: 1.0385x; 1.0385x over previous
import functools

import jax
import jax.numpy as jnp
from jax import lax
from jax.experimental import pallas as pl
from jax.experimental.pallas import tpu as pltpu
from jax.experimental.pallas import tpu_sc as plsc

_NOBS = 1000000
_ROWS = 16384
_COLS = 200
_NC = 2
_NS = 16
_NW = _NC * _NS
_RPW = _ROWS // _NW
_RCH = 16
_CHUNK = _RCH * _COLS
_NCHUNK = _RPW // _RCH
_STAGE = 62496
_SCH = 10416
_L = 16
_TC0 = _COLS - _L

_mesh = plsc.VectorSubcoreMesh(core_axis_name="c", subcore_axis_name="s")


@functools.partial(
    pl.kernel,
    out_type=jax.ShapeDtypeStruct((_ROWS, _COLS), jnp.float32),
    mesh=_mesh,
    compiler_params=pltpu.CompilerParams(needs_layout_passes=False),
    scratch_types=[
        pltpu.VMEM_SHARED((_NOBS,), jnp.float32),
        [pltpu.VMEM((_RCH, _COLS), jnp.int32) for _ in range(2)],
        [pltpu.VMEM((_CHUNK,), jnp.int32) for _ in range(2)],
        [pltpu.VMEM((_CHUNK,), jnp.float32) for _ in range(2)],
        [pltpu.VMEM((_RCH, _COLS), jnp.float32) for _ in range(2)],
        pltpu.VMEM((_SCH,), jnp.float32),
        pltpu.SemaphoreType.DMA((2,)),
        pltpu.SemaphoreType.DMA((2,)),
        pltpu.SemaphoreType.DMA((2,)),
    ],
)
def _gather_kernel(idx_hbm, table_hbm, out_hbm, tbl_s, idx2_v, idx1_v,
                   out1_v, out2_v, stg_v, lsem, gsem, ssem):
    s = lax.axis_index("s")
    wid = s * _NC + lax.axis_index("c")
    base = wid * _RPW
    lane = lax.iota(jnp.int32, _L)

    def ichunk(i):
        return idx_hbm.at[pl.ds(base + i * _RCH, _RCH), :]

    def ochunk(i):
        return out_hbm.at[pl.ds(base + i * _RCH, _RCH), :]

    def compact(src2, dst1):
        def body(r, carry):
            for c in range(0, _TC0, _L):
                dst1[pl.ds(r * _COLS + c, _L)] = src2[r, pl.ds(c, _L)]
            rv = jnp.full((_L,), r, dtype=jnp.int32)
            tail = plsc.load_gather(src2, [rv, lane + _TC0])
            dst1[pl.ds(r * _COLS + _TC0, _L)] = tail
            return carry
        lax.fori_loop(0, _RCH, body, 0)

    def expand(src1, dst2):
        def body(r, carry):
            for c in range(0, _TC0, _L):
                dst2[r, pl.ds(c, _L)] = src1[pl.ds(r * _COLS + c, _L)]
            rv = jnp.full((_L,), r, dtype=jnp.int32)
            tail = src1[pl.ds(r * _COLS + _TC0, _L)]
            plsc.store_scatter(dst2, [rv, lane + _TC0], tail)
            return carry
        lax.fori_loop(0, _RCH, body, 0)

    def drain_gather(b):
        pltpu.make_async_copy(
            table_hbm.at[pl.ds(0, _CHUNK)], out1_v[b], gsem.at[b]).wait()

    pltpu.async_copy(ichunk(0), idx2_v[0], lsem.at[0])

    for j in range(_STAGE // _SCH):
        soff = s * _STAGE + j * _SCH
        pltpu.sync_copy(table_hbm.at[pl.ds(soff, _SCH)], stg_v)
        pltpu.sync_copy(stg_v, tbl_s.at[pl.ds(soff, _SCH)])

    @pl.when(s == 0)
    def _():
        rem = _NOBS - _NS * _STAGE
        roff = _NS * _STAGE
        bv = stg_v.at[pl.ds(0, rem)]
        pltpu.sync_copy(table_hbm.at[pl.ds(roff, rem)], bv)
        pltpu.sync_copy(bv, tbl_s.at[pl.ds(roff, rem)])

    plsc.subcore_barrier()

    pltpu.async_copy(ichunk(1), idx2_v[1], lsem.at[1])
    pltpu.make_async_copy(ichunk(0), idx2_v[0], lsem.at[0]).wait()
    compact(idx2_v[0], idx1_v[0])

    for i in range(_NCHUNK):
        b = i % 2
        pltpu.async_copy(tbl_s.at[idx1_v[b]], out1_v[b], gsem.at[b])
        if i + 2 < _NCHUNK:
            pltpu.async_copy(ichunk(i + 2), idx2_v[b], lsem.at[b])
        if i >= 1:
            drain_gather(1 - b)
            if i >= 3:
                pltpu.make_async_copy(
                    out2_v[1 - b], ochunk(i - 3), ssem.at[1 - b]).wait()
            expand(out1_v[1 - b], out2_v[1 - b])
            pltpu.async_copy(out2_v[1 - b], ochunk(i - 1), ssem.at[1 - b])
        if i + 1 < _NCHUNK:
            pltpu.make_async_copy(
                ichunk(i + 1), idx2_v[1 - b], lsem.at[1 - b]).wait()
            compact(idx2_v[1 - b], idx1_v[1 - b])

    last = (_NCHUNK - 1) % 2
    drain_gather(last)
    pltpu.make_async_copy(
        out2_v[last], ochunk(_NCHUNK - 3), ssem.at[last]).wait()
    expand(out1_v[last], out2_v[last])
    pltpu.async_copy(out2_v[last], ochunk(_NCHUNK - 1), ssem.at[last])
    pltpu.make_async_copy(
        out2_v[1 - last], ochunk(_NCHUNK - 2), ssem.at[1 - last]).wait()
    pltpu.make_async_copy(
        out2_v[last], ochunk(_NCHUNK - 1), ssem.at[last]).wait()


def kernel(iobs, cloud_fraction_delta):
    return _gather_kernel(iobs, cloud_fraction_delta)
